# baseline (speedup 1.0000x reference)
import functools

import jax
import jax.numpy as jnp
from jax import lax
from jax.experimental import pallas as pl
from jax.experimental.pallas import tpu as pltpu
from jax.experimental.pallas import tpu_sc as plsc

_N = 10000
_E = 320000
_H = 128
_G = 64
_NC = 2
_NS = 16
_NW = _NC * _NS
_EPW = _E // _NW
_CH = 80
_NCHUNK = _EPW // _CH
_NBUF = 3
_NIDX = 8
_PERIOD = 24
_MAIN = (_NCHUNK // _PERIOD) * _PERIOD
_RPT = 624
_TAIL = _N - _NS * _RPT
_RB = 1000
_NRB = _N // _RB


@functools.cache
def _make_sc_segsum():
    mesh = plsc.VectorSubcoreMesh(core_axis_name="c", subcore_axis_name="s")
    out_t = (jax.ShapeDtypeStruct((_N, _H), jnp.float32),
             jax.ShapeDtypeStruct((_N, _H), jnp.float32))

    @functools.partial(
        pl.kernel, mesh=mesh, out_type=out_t,
        scratch_types=[
            pltpu.VMEM((_NIDX, _CH), jnp.int32),
            pltpu.VMEM((_NIDX, _CH), jnp.int32),
            pltpu.VMEM((_NBUF, _CH, _H), jnp.float32),
            pltpu.VMEM((8, _H), jnp.float32),
            pltpu.VMEM_SHARED((_N, _H), jnp.float32),
        ] + [pltpu.SemaphoreType.DMA] * (_NBUF + _NIDX))
    def seg(h_hbm, src_hbm, dst_hbm, out0, out1, sidx, didx, rows_v, zbuf,
            acc, *sems):
        gsems = sems[:_NBUF]
        isems = sems[_NBUF:]
        cid = lax.axis_index("c")
        sid = lax.axis_index("s")
        wid = cid * _NS + sid

        def issue_idx(ch, slot):
            e0 = wid * _EPW + ch * _CH
            pltpu.async_copy(src_hbm.at[pl.ds(e0, _CH)], sidx.at[slot],
                             isems[slot])
            pltpu.async_copy(dst_hbm.at[pl.ds(e0, _CH)], didx.at[slot],
                             isems[slot])

        def wait_idx(slot):
            pltpu.make_async_copy(src_hbm.at[pl.ds(0, _CH)], sidx.at[slot],
                                  isems[slot]).wait()
            pltpu.make_async_copy(dst_hbm.at[pl.ds(0, _CH)], didx.at[slot],
                                  isems[slot]).wait()

        def issue_gather(slot, b):
            pltpu.async_copy(h_hbm.at[sidx.at[slot]], rows_v.at[b], gsems[b])

        def wait_gather(b):
            pltpu.make_async_copy(h_hbm.at[sidx.at[0]], rows_v.at[b],
                                  gsems[b]).wait()

        for s in range(_NIDX - 2):
            issue_idx(s, s)
        for b in range(2):
            wait_idx(b)
            issue_gather(b, b)

        for i in range(8):
            for j in range(_H // 16):
                zbuf[i, pl.ds(j * 16, 16)] = jnp.zeros((16,), jnp.float32)
        base_r = sid * _RPT

        def zacc(j, c):
            pltpu.sync_copy(zbuf, acc.at[pl.ds(base_r + j * 8, 8)])
            return c
        lax.fori_loop(0, _RPT // 8, zacc, 0)

        @pl.when(sid == _NS - 1)
        def _():
            pltpu.sync_copy(zbuf.at[pl.ds(0, _TAIL)],
                            acc.at[pl.ds(_NS * _RPT, _TAIL)])
        plsc.subcore_barrier()

        def step(j, c):
            for k in range(_PERIOD):
                i = j * _PERIOD + k
                wait_gather(k % _NBUF)
                pltpu.sync_copy(rows_v.at[k % _NBUF],
                                acc.at[didx.at[k % _NIDX]], add=True)
                wait_idx((k + 2) % _NIDX)
                issue_gather((k + 2) % _NIDX, (k + 2) % _NBUF)

                @pl.when(i + 6 < _NCHUNK)
                def _():
                    issue_idx(i + 6, (k + 6) % _NIDX)
            return c
        lax.fori_loop(0, _NCHUNK // _PERIOD, step, 0)
        for k in range(_MAIN, _NCHUNK):
            wait_gather(k % _NBUF)
            pltpu.sync_copy(rows_v.at[k % _NBUF],
                            acc.at[didx.at[k % _NIDX]], add=True)
            if k + 2 < _NCHUNK:
                wait_idx((k + 2) % _NIDX)
                issue_gather((k + 2) % _NIDX, (k + 2) % _NBUF)
        plsc.subcore_barrier()

        @pl.when(cid == 0)
        def _():
            pltpu.sync_copy(acc.at[pl.ds(base_r, _RPT)], out0.at[pl.ds(base_r, _RPT)])

            @pl.when(sid == _NS - 1)
            def _():
                pltpu.sync_copy(acc.at[pl.ds(_NS * _RPT, _TAIL)],
                                out0.at[pl.ds(_NS * _RPT, _TAIL)])

        @pl.when(cid == 1)
        def _():
            pltpu.sync_copy(acc.at[pl.ds(base_r, _RPT)], out1.at[pl.ds(base_r, _RPT)])

            @pl.when(sid == _NS - 1)
            def _():
                pltpu.sync_copy(acc.at[pl.ds(_NS * _RPT, _TAIL)],
                                out1.at[pl.ds(_NS * _RPT, _TAIL)])

    return seg


def _stats_to_affine(st_ref, g_row, b_row):
    mu = st_ref[0:1, :] * (1.0 / _N)
    var = st_ref[1:2, :] * (1.0 / _N) - mu * mu
    a = g_row * lax.rsqrt(var + 1e-5)
    return a, b_row - mu * a


def _mlp_body(h_ref, p0_ref, p1_ref, sc_ref, w1_ref, b1_ref, w2_ref, b2_ref,
              z2_ref, st_ref):
    z = h_ref[...] * sc_ref[...] + (p0_ref[...] + p1_ref[...])
    z1 = jnp.maximum(
        jnp.dot(z, w1_ref[0], preferred_element_type=jnp.float32) + b1_ref[0], 0.0)
    z2 = jnp.maximum(
        jnp.dot(z1, w2_ref[0], preferred_element_type=jnp.float32) + b2_ref[0], 0.0)
    z2_ref[...] = z2
    s = jnp.sum(z2, axis=0, keepdims=True)
    ss = jnp.sum(z2 * z2, axis=0, keepdims=True)
    upd = jnp.concatenate([s, ss, jnp.zeros((6, _H), jnp.float32)], axis=0)

    @pl.when(pl.program_id(0) == 0)
    def _():
        st_ref[...] = jnp.zeros_like(st_ref)

    st_ref[...] += upd


@functools.cache
def _mlp_call(l):
    row = pl.BlockSpec((_RB, _H), lambda i: (i, 0))
    lrow = pl.BlockSpec((1, 1, _H), lambda i: (l, 0, 0))
    return pl.pallas_call(
        _mlp_body,
        grid=(_NRB,),
        in_specs=[
            row, row, row,
            pl.BlockSpec((1, _H), lambda i: (0, 0)),
            pl.BlockSpec((1, _H, _H), lambda i: (l, 0, 0)),
            lrow,
            pl.BlockSpec((1, _H, _H), lambda i: (l, 0, 0)),
            lrow,
        ],
        out_specs=[row, pl.BlockSpec((8, _H), lambda i: (0, 0))],
        out_shape=[
            jax.ShapeDtypeStruct((_N, _H), jnp.float32),
            jax.ShapeDtypeStruct((8, _H), jnp.float32),
        ],
        compiler_params=pltpu.CompilerParams(
            dimension_semantics=("arbitrary",)),
    )


def _aff_body(z_ref, st_ref, g_ref, bt_ref, o_ref):
    a, b = _stats_to_affine(st_ref, g_ref[0], bt_ref[0])
    o_ref[...] = z_ref[...] * a + b


@functools.cache
def _aff_call(l):
    lrow = pl.BlockSpec((1, 1, _H), lambda i: (l, 0, 0))
    return pl.pallas_call(
        _aff_body,
        grid=(_NRB,),
        in_specs=[
            pl.BlockSpec((_RB, _H), lambda i: (i, 0)),
            pl.BlockSpec((8, _H), lambda i: (0, 0)),
            lrow,
            lrow,
        ],
        out_specs=pl.BlockSpec((_RB, _H), lambda i: (i, 0)),
        out_shape=jax.ShapeDtypeStruct((_N, _H), jnp.float32),
        compiler_params=pltpu.CompilerParams(
            dimension_semantics=("arbitrary",)),
    )


def _pool_body(z_ref, bt3_ref, st_ref, g_ref, be_ref, w1_ref, b1_ref,
               w2r_ref, b2_ref, o_ref, accp, accc):
    i = pl.program_id(0)

    @pl.when(i == 0)
    def _():
        accp[...] = jnp.zeros_like(accp)
        accc[...] = jnp.zeros_like(accc)

    bt = bt3_ref[0]
    gi = lax.broadcasted_iota(jnp.int32, (_G, _RB), 0)
    oh = (gi == bt).astype(jnp.float32)
    accp[...] += lax.dot_general(oh, z_ref[...], (((1,), (0,)), ((), ())),
                                 preferred_element_type=jnp.float32)
    accc[...] += jnp.broadcast_to(jnp.sum(oh, axis=1, keepdims=True), (_G, _H))

    @pl.when(i == pl.num_programs(0) - 1)
    def _():
        a, b = _stats_to_affine(st_ref, g_ref[0], be_ref[0])
        pooled = accp[...] / jnp.maximum(accc[...], 1.0)
        pooled = pooled * a + b
        r1 = jnp.maximum(
            jnp.dot(pooled, w1_ref[...], preferred_element_type=jnp.float32) + b1_ref[...], 0.0)
        o_ref[...] = jnp.sum(r1 * w2r_ref[...], axis=1, keepdims=True) + b2_ref[...]


@functools.cache
def _pool_call(l):
    lrow = pl.BlockSpec((1, 1, _H), lambda i: (l, 0, 0))
    return pl.pallas_call(
        _pool_body,
        grid=(_NRB,),
        in_specs=[
            pl.BlockSpec((_RB, _H), lambda i: (i, 0)),
            pl.BlockSpec((1, 1, _RB), lambda i: (i, 0, 0)),
            pl.BlockSpec((8, _H), lambda i: (0, 0)),
            lrow,
            lrow,
            pl.BlockSpec((_H, _H), lambda i: (0, 0)),
            pl.BlockSpec((1, _H), lambda i: (0, 0)),
            pl.BlockSpec((1, _H), lambda i: (0, 0)),
            pl.BlockSpec((1, 1), lambda i: (0, 0)),
        ],
        out_specs=pl.BlockSpec((_G, 1), lambda i: (0, 0)),
        out_shape=jax.ShapeDtypeStruct((_G, 1), jnp.float32),
        scratch_shapes=[
            pltpu.VMEM((_G, _H), jnp.float32),
            pltpu.VMEM((_G, _H), jnp.float32),
        ],
        compiler_params=pltpu.CompilerParams(
            dimension_semantics=("arbitrary",)),
    )


def kernel(x, edge_index, batch, W1s, b1s, W2s, b2s, gammas, betas, eps_vec,
           lin1_W, lin1_b, lin2_W, lin2_b):
    src = edge_index[0]
    dst = edge_index[1]
    batch3 = batch.reshape(_NRB, 1, _RB)
    scal_rows = (1.0 + eps_vec)[:, None] * jnp.ones((1, _H), jnp.float32)
    num_layers = W1s.shape[0]
    b1s3 = b1s.reshape(num_layers, 1, _H)
    b2s3 = b2s.reshape(num_layers, 1, _H)
    gam3 = gammas.reshape(num_layers, 1, _H)
    bet3 = betas.reshape(num_layers, 1, _H)

    h = x
    out = None
    for l in range(num_layers):
        p0, p1 = _make_sc_segsum()(h, src, dst)
        z2, st = _mlp_call(l)(h, p0, p1, scal_rows[l:l + 1], W1s, b1s3, W2s,
                              b2s3)
        if l < num_layers - 1:
            h = _aff_call(l)(z2, st, gam3, bet3)
        else:
            out = _pool_call(l)(z2, batch3, st, gam3, bet3,
                                lin1_W, lin1_b.reshape(1, _H),
                                lin2_W.reshape(1, _H), lin2_b.reshape(1, 1))
    return out


# speedup vs baseline: 1.0004x; 1.0004x over previous
import functools

import jax
import jax.numpy as jnp
from jax import lax
from jax.experimental import pallas as pl
from jax.experimental.pallas import tpu as pltpu
from jax.experimental.pallas import tpu_sc as plsc

_N = 10000
_E = 320000
_H = 128
_G = 64
_NC = 2
_NS = 16
_NW = _NC * _NS
_EPW = _E // _NW
_CH = 80
_NCHUNK = _EPW // _CH
_NBUF = 3
_NIDX = 8
_PERIOD = 24
_MAIN = (_NCHUNK // _PERIOD) * _PERIOD
_RPT = 624
_TAIL = _N - _NS * _RPT
_RB = 1000
_NRB = _N // _RB


@functools.cache
def _make_sc_segsum():
    mesh = plsc.VectorSubcoreMesh(core_axis_name="c", subcore_axis_name="s")
    out_t = (jax.ShapeDtypeStruct((_N, _H), jnp.float32),
             jax.ShapeDtypeStruct((_N, _H), jnp.float32))

    @functools.partial(
        pl.kernel, mesh=mesh, out_type=out_t,
        scratch_types=[
            pltpu.VMEM((_NIDX, _CH), jnp.int32),
            pltpu.VMEM((_NIDX, _CH), jnp.int32),
            pltpu.VMEM((_NBUF, _CH, _H), jnp.float32),
            pltpu.VMEM((8, _H), jnp.float32),
            pltpu.VMEM_SHARED((_N, _H), jnp.float32),
        ] + [pltpu.SemaphoreType.DMA] * (_NBUF + _NIDX))
    def seg(h_hbm, src_hbm, dst_hbm, out0, out1, sidx, didx, rows_v, zbuf,
            acc, *sems):
        gsems = sems[:_NBUF]
        isems = sems[_NBUF:]
        cid = lax.axis_index("c")
        sid = lax.axis_index("s")
        wid = cid * _NS + sid

        def issue_idx(ch, slot):
            e0 = wid * _EPW + ch * _CH
            pltpu.async_copy(src_hbm.at[pl.ds(e0, _CH)], sidx.at[slot],
                             isems[slot])
            pltpu.async_copy(dst_hbm.at[pl.ds(e0, _CH)], didx.at[slot],
                             isems[slot])

        def wait_idx(slot):
            pltpu.make_async_copy(src_hbm.at[pl.ds(0, _CH)], sidx.at[slot],
                                  isems[slot]).wait()
            pltpu.make_async_copy(dst_hbm.at[pl.ds(0, _CH)], didx.at[slot],
                                  isems[slot]).wait()

        def issue_gather(slot, b):
            pltpu.async_copy(h_hbm.at[sidx.at[slot]], rows_v.at[b], gsems[b])

        def wait_gather(b):
            pltpu.make_async_copy(h_hbm.at[sidx.at[0]], rows_v.at[b],
                                  gsems[b]).wait()

        for s in range(_NIDX - 2):
            issue_idx(s, s)
        for b in range(2):
            wait_idx(b)
            issue_gather(b, b)

        for i in range(8):
            for j in range(_H // 16):
                zbuf[i, pl.ds(j * 16, 16)] = jnp.zeros((16,), jnp.float32)
        base_r = sid * _RPT

        def zacc(j, c):
            pltpu.sync_copy(zbuf, acc.at[pl.ds(base_r + j * 8, 8)])
            return c
        lax.fori_loop(0, _RPT // 8, zacc, 0)

        @pl.when(sid == _NS - 1)
        def _():
            pltpu.sync_copy(zbuf.at[pl.ds(0, _TAIL)],
                            acc.at[pl.ds(_NS * _RPT, _TAIL)])
        plsc.subcore_barrier()

        def step(j, c):
            for k in range(_PERIOD):
                i = j * _PERIOD + k
                wait_gather(k % _NBUF)
                pltpu.sync_copy(rows_v.at[k % _NBUF],
                                acc.at[didx.at[k % _NIDX]], add=True)
                wait_idx((k + 2) % _NIDX)
                issue_gather((k + 2) % _NIDX, (k + 2) % _NBUF)

                @pl.when(i + 6 < _NCHUNK)
                def _():
                    issue_idx(i + 6, (k + 6) % _NIDX)
            return c
        lax.fori_loop(0, _NCHUNK // _PERIOD, step, 0)
        for k in range(_MAIN, _NCHUNK):
            wait_gather(k % _NBUF)
            pltpu.sync_copy(rows_v.at[k % _NBUF],
                            acc.at[didx.at[k % _NIDX]], add=True)
            if k + 2 < _NCHUNK:
                wait_idx((k + 2) % _NIDX)
                issue_gather((k + 2) % _NIDX, (k + 2) % _NBUF)
        plsc.subcore_barrier()

        @pl.when(cid == 0)
        def _():
            pltpu.sync_copy(acc.at[pl.ds(base_r, _RPT)], out0.at[pl.ds(base_r, _RPT)])

            @pl.when(sid == _NS - 1)
            def _():
                pltpu.sync_copy(acc.at[pl.ds(_NS * _RPT, _TAIL)],
                                out0.at[pl.ds(_NS * _RPT, _TAIL)])

        @pl.when(cid == 1)
        def _():
            pltpu.sync_copy(acc.at[pl.ds(base_r, _RPT)], out1.at[pl.ds(base_r, _RPT)])

            @pl.when(sid == _NS - 1)
            def _():
                pltpu.sync_copy(acc.at[pl.ds(_NS * _RPT, _TAIL)],
                                out1.at[pl.ds(_NS * _RPT, _TAIL)])

    return seg


def _stats_to_affine(st_ref, g_row, b_row):
    mu = st_ref[0:1, :] * (1.0 / _N)
    var = st_ref[1:2, :] * (1.0 / _N) - mu * mu
    a = g_row * lax.rsqrt(var + 1e-5)
    return a, b_row - mu * a


def _mlp_body(h_ref, p0_ref, p1_ref, sc_ref, w1_ref, b1_ref, w2_ref, b2_ref,
              z2_ref, st_ref):
    z = h_ref[...] * sc_ref[...] + (p0_ref[...] + p1_ref[...])
    z1 = jnp.maximum(
        jnp.dot(z, w1_ref[0], preferred_element_type=jnp.float32) + b1_ref[0], 0.0)
    z2 = jnp.maximum(
        jnp.dot(z1, w2_ref[0], preferred_element_type=jnp.float32) + b2_ref[0], 0.0)
    z2_ref[...] = z2
    s = jnp.sum(z2, axis=0, keepdims=True)
    ss = jnp.sum(z2 * z2, axis=0, keepdims=True)
    upd = jnp.concatenate([s, ss, jnp.zeros((6, _H), jnp.float32)], axis=0)

    @pl.when(pl.program_id(0) == 0)
    def _():
        st_ref[...] = jnp.zeros_like(st_ref)

    st_ref[...] += upd


@functools.cache
def _mlp_call(l):
    del l
    row = pl.BlockSpec((_RB, _H), lambda i: (i, 0))
    lrow = pl.BlockSpec((1, 1, _H), lambda i: (0, 0, 0))
    return pl.pallas_call(
        _mlp_body,
        grid=(_NRB,),
        in_specs=[
            row, row, row,
            pl.BlockSpec((1, _H), lambda i: (0, 0)),
            pl.BlockSpec((1, _H, _H), lambda i: (0, 0, 0)),
            lrow,
            pl.BlockSpec((1, _H, _H), lambda i: (0, 0, 0)),
            lrow,
        ],
        out_specs=[row, pl.BlockSpec((8, _H), lambda i: (0, 0))],
        out_shape=[
            jax.ShapeDtypeStruct((_N, _H), jnp.float32),
            jax.ShapeDtypeStruct((8, _H), jnp.float32),
        ],
        compiler_params=pltpu.CompilerParams(
            dimension_semantics=("arbitrary",)),
    )


def _aff_body(z_ref, st_ref, g_ref, bt_ref, o_ref):
    a, b = _stats_to_affine(st_ref, g_ref[0], bt_ref[0])
    o_ref[...] = z_ref[...] * a + b


@functools.cache
def _aff_call(l):
    del l
    lrow = pl.BlockSpec((1, 1, _H), lambda i: (0, 0, 0))
    return pl.pallas_call(
        _aff_body,
        grid=(_NRB,),
        in_specs=[
            pl.BlockSpec((_RB, _H), lambda i: (i, 0)),
            pl.BlockSpec((8, _H), lambda i: (0, 0)),
            lrow,
            lrow,
        ],
        out_specs=pl.BlockSpec((_RB, _H), lambda i: (i, 0)),
        out_shape=jax.ShapeDtypeStruct((_N, _H), jnp.float32),
        compiler_params=pltpu.CompilerParams(
            dimension_semantics=("arbitrary",)),
    )


def _pool_body(z_ref, bt3_ref, st_ref, g_ref, be_ref, w1_ref, b1_ref,
               w2r_ref, b2_ref, o_ref, accp, accc):
    i = pl.program_id(0)

    @pl.when(i == 0)
    def _():
        accp[...] = jnp.zeros_like(accp)
        accc[...] = jnp.zeros_like(accc)

    bt = bt3_ref[0]
    gi = lax.broadcasted_iota(jnp.int32, (_G, _RB), 0)
    oh = (gi == bt).astype(jnp.float32)
    accp[...] += lax.dot_general(oh, z_ref[...], (((1,), (0,)), ((), ())),
                                 preferred_element_type=jnp.float32)
    accc[...] += jnp.broadcast_to(jnp.sum(oh, axis=1, keepdims=True), (_G, _H))

    @pl.when(i == pl.num_programs(0) - 1)
    def _():
        a, b = _stats_to_affine(st_ref, g_ref[0], be_ref[0])
        pooled = accp[...] / jnp.maximum(accc[...], 1.0)
        pooled = pooled * a + b
        r1 = jnp.maximum(
            jnp.dot(pooled, w1_ref[...], preferred_element_type=jnp.float32) + b1_ref[...], 0.0)
        o_ref[...] = jnp.sum(r1 * w2r_ref[...], axis=1, keepdims=True) + b2_ref[...]


@functools.cache
def _pool_call(l):
    del l
    lrow = pl.BlockSpec((1, 1, _H), lambda i: (0, 0, 0))
    return pl.pallas_call(
        _pool_body,
        grid=(_NRB,),
        in_specs=[
            pl.BlockSpec((_RB, _H), lambda i: (i, 0)),
            pl.BlockSpec((1, 1, _RB), lambda i: (i, 0, 0)),
            pl.BlockSpec((8, _H), lambda i: (0, 0)),
            lrow,
            lrow,
            pl.BlockSpec((_H, _H), lambda i: (0, 0)),
            pl.BlockSpec((1, _H), lambda i: (0, 0)),
            pl.BlockSpec((1, _H), lambda i: (0, 0)),
            pl.BlockSpec((1, 1), lambda i: (0, 0)),
        ],
        out_specs=pl.BlockSpec((_G, 1), lambda i: (0, 0)),
        out_shape=jax.ShapeDtypeStruct((_G, 1), jnp.float32),
        scratch_shapes=[
            pltpu.VMEM((_G, _H), jnp.float32),
            pltpu.VMEM((_G, _H), jnp.float32),
        ],
        compiler_params=pltpu.CompilerParams(
            dimension_semantics=("arbitrary",)),
    )


def kernel(x, edge_index, batch, W1s, b1s, W2s, b2s, gammas, betas, eps_vec,
           lin1_W, lin1_b, lin2_W, lin2_b):
    src = edge_index[0]
    dst = edge_index[1]
    batch3 = batch.reshape(_NRB, 1, _RB)
    scal_rows = (1.0 + eps_vec)[:, None] * jnp.ones((1, _H), jnp.float32)
    num_layers = W1s.shape[0]
    b1s3 = b1s.reshape(num_layers, 1, _H)
    b2s3 = b2s.reshape(num_layers, 1, _H)
    gam3 = gammas.reshape(num_layers, 1, _H)
    bet3 = betas.reshape(num_layers, 1, _H)

    h = x
    out = None
    for l in range(num_layers):
        p0, p1 = _make_sc_segsum()(h, src, dst)
        z2, st = _mlp_call(0)(h, p0, p1, scal_rows[l:l + 1], W1s[l:l + 1],
                              b1s3[l:l + 1], W2s[l:l + 1], b2s3[l:l + 1])
        if l < num_layers - 1:
            h = _aff_call(0)(z2, st, gam3[l:l + 1], bet3[l:l + 1])
        else:
            out = _pool_call(0)(z2, batch3, st, gam3[l:l + 1], bet3[l:l + 1],
                                lin1_W, lin1_b.reshape(1, _H),
                                lin2_W.reshape(1, _H), lin2_b.reshape(1, 1))
    return out


# speedup vs baseline: 1.1762x; 1.1757x over previous
import functools

import jax
import jax.numpy as jnp
from jax import lax
from jax.experimental import pallas as pl
from jax.experimental.pallas import tpu as pltpu
from jax.experimental.pallas import tpu_sc as plsc

_N = 10000
_E = 320000
_H = 128
_G = 64
_NC = 2
_NS = 16
_NW = _NC * _NS
_EPW = _E // _NW
_CH = 80
_NCHUNK = _EPW // _CH
_NBUF = 3
_NIDX = 8
_PERIOD = 24
_MAIN = (_NCHUNK // _PERIOD) * _PERIOD
_RPT = 624
_TAIL = _N - _NS * _RPT
_RB = 1000
_NRB = _N // _RB


@functools.cache
def _make_sc_segsum():
    mesh = plsc.VectorSubcoreMesh(core_axis_name="c", subcore_axis_name="s")
    out_t = (jax.ShapeDtypeStruct((_N, _H), jnp.float32),
             jax.ShapeDtypeStruct((_N, _H), jnp.float32))

    @functools.partial(
        pl.kernel, mesh=mesh, out_type=out_t,
        scratch_types=[
            pltpu.VMEM((_NIDX, _CH), jnp.int32),
            pltpu.VMEM((_NIDX, _CH), jnp.int32),
            pltpu.VMEM((_NBUF, _CH, _H), jnp.float32),
            pltpu.VMEM((8, _H), jnp.float32),
            pltpu.VMEM_SHARED((_N, _H), jnp.float32),
        ] + [pltpu.SemaphoreType.DMA] * (_NBUF + _NIDX))
    def seg(h_hbm, src_hbm, dst_hbm, out0, out1, sidx, didx, rows_v, zbuf,
            acc, *sems):
        gsems = sems[:_NBUF]
        isems = sems[_NBUF:]
        cid = lax.axis_index("c")
        sid = lax.axis_index("s")
        wid = cid * _NS + sid

        def issue_idx(ch, slot):
            e0 = wid * _EPW + ch * _CH
            pltpu.async_copy(src_hbm.at[pl.ds(e0, _CH)], sidx.at[slot],
                             isems[slot])
            pltpu.async_copy(dst_hbm.at[pl.ds(e0, _CH)], didx.at[slot],
                             isems[slot])

        def wait_idx(slot):
            pltpu.make_async_copy(src_hbm.at[pl.ds(0, _CH)], sidx.at[slot],
                                  isems[slot]).wait()
            pltpu.make_async_copy(dst_hbm.at[pl.ds(0, _CH)], didx.at[slot],
                                  isems[slot]).wait()

        def issue_gather(slot, b):
            pltpu.async_copy(h_hbm.at[sidx.at[slot]], rows_v.at[b], gsems[b])

        def wait_gather(b):
            pltpu.make_async_copy(h_hbm.at[sidx.at[0]], rows_v.at[b],
                                  gsems[b]).wait()

        for s in range(_NIDX - 2):
            issue_idx(s, s)
        for b in range(_NBUF):
            wait_idx(b)
            issue_gather(b, b)

        for i in range(8):
            for j in range(_H // 16):
                zbuf[i, pl.ds(j * 16, 16)] = jnp.zeros((16,), jnp.float32)
        base_r = sid * _RPT

        def zacc(j, c):
            pltpu.sync_copy(zbuf, acc.at[pl.ds(base_r + j * 8, 8)])
            return c
        lax.fori_loop(0, _RPT // 8, zacc, 0)

        @pl.when(sid == _NS - 1)
        def _():
            pltpu.sync_copy(zbuf.at[pl.ds(0, _TAIL)],
                            acc.at[pl.ds(_NS * _RPT, _TAIL)])
        plsc.subcore_barrier()

        def step(j, c):
            for k in range(_PERIOD):
                i = j * _PERIOD + k
                wait_gather(k % _NBUF)
                pltpu.sync_copy(rows_v.at[k % _NBUF],
                                acc.at[didx.at[k % _NIDX]], add=True)
                wait_idx((k + 3) % _NIDX)
                issue_gather((k + 3) % _NIDX, k % _NBUF)

                @pl.when(i + 6 < _NCHUNK)
                def _():
                    issue_idx(i + 6, (k + 6) % _NIDX)
            return c
        lax.fori_loop(0, _NCHUNK // _PERIOD, step, 0)
        for k in range(_MAIN, _NCHUNK):
            wait_gather(k % _NBUF)
            pltpu.sync_copy(rows_v.at[k % _NBUF],
                            acc.at[didx.at[k % _NIDX]], add=True)
            if k + 3 < _NCHUNK:
                wait_idx((k + 3) % _NIDX)
                issue_gather((k + 3) % _NIDX, k % _NBUF)
        plsc.subcore_barrier()

        @pl.when(cid == 0)
        def _():
            pltpu.sync_copy(acc.at[pl.ds(base_r, _RPT)], out0.at[pl.ds(base_r, _RPT)])

            @pl.when(sid == _NS - 1)
            def _():
                pltpu.sync_copy(acc.at[pl.ds(_NS * _RPT, _TAIL)],
                                out0.at[pl.ds(_NS * _RPT, _TAIL)])

        @pl.when(cid == 1)
        def _():
            pltpu.sync_copy(acc.at[pl.ds(base_r, _RPT)], out1.at[pl.ds(base_r, _RPT)])

            @pl.when(sid == _NS - 1)
            def _():
                pltpu.sync_copy(acc.at[pl.ds(_NS * _RPT, _TAIL)],
                                out1.at[pl.ds(_NS * _RPT, _TAIL)])

    return seg


def _stats_to_affine(st_ref, g_row, b_row):
    mu = st_ref[0:1, :] * (1.0 / _N)
    var = st_ref[1:2, :] * (1.0 / _N) - mu * mu
    a = g_row * lax.rsqrt(var + 1e-5)
    return a, b_row - mu * a


def _mlp_body(h_ref, p0_ref, p1_ref, sc_ref, w1_ref, b1_ref, w2_ref, b2_ref,
              z2_ref, st_ref):
    z = h_ref[...] * sc_ref[...] + (p0_ref[...] + p1_ref[...])
    z1 = jnp.maximum(
        jnp.dot(z, w1_ref[0], preferred_element_type=jnp.float32) + b1_ref[0], 0.0)
    z2 = jnp.maximum(
        jnp.dot(z1, w2_ref[0], preferred_element_type=jnp.float32) + b2_ref[0], 0.0)
    z2_ref[...] = z2
    s = jnp.sum(z2, axis=0, keepdims=True)
    ss = jnp.sum(z2 * z2, axis=0, keepdims=True)
    upd = jnp.concatenate([s, ss, jnp.zeros((6, _H), jnp.float32)], axis=0)

    @pl.when(pl.program_id(0) == 0)
    def _():
        st_ref[...] = jnp.zeros_like(st_ref)

    st_ref[...] += upd


@functools.cache
def _mlp_call(l):
    del l
    row = pl.BlockSpec((_RB, _H), lambda i: (i, 0))
    lrow = pl.BlockSpec((1, 1, _H), lambda i: (0, 0, 0))
    return pl.pallas_call(
        _mlp_body,
        grid=(_NRB,),
        in_specs=[
            row, row, row,
            pl.BlockSpec((1, _H), lambda i: (0, 0)),
            pl.BlockSpec((1, _H, _H), lambda i: (0, 0, 0)),
            lrow,
            pl.BlockSpec((1, _H, _H), lambda i: (0, 0, 0)),
            lrow,
        ],
        out_specs=[row, pl.BlockSpec((8, _H), lambda i: (0, 0))],
        out_shape=[
            jax.ShapeDtypeStruct((_N, _H), jnp.float32),
            jax.ShapeDtypeStruct((8, _H), jnp.float32),
        ],
        compiler_params=pltpu.CompilerParams(
            dimension_semantics=("arbitrary",)),
    )


def _aff_body(z_ref, st_ref, g_ref, bt_ref, o_ref):
    a, b = _stats_to_affine(st_ref, g_ref[0], bt_ref[0])
    o_ref[...] = z_ref[...] * a + b


@functools.cache
def _aff_call(l):
    del l
    lrow = pl.BlockSpec((1, 1, _H), lambda i: (0, 0, 0))
    return pl.pallas_call(
        _aff_body,
        grid=(_NRB,),
        in_specs=[
            pl.BlockSpec((_RB, _H), lambda i: (i, 0)),
            pl.BlockSpec((8, _H), lambda i: (0, 0)),
            lrow,
            lrow,
        ],
        out_specs=pl.BlockSpec((_RB, _H), lambda i: (i, 0)),
        out_shape=jax.ShapeDtypeStruct((_N, _H), jnp.float32),
        compiler_params=pltpu.CompilerParams(
            dimension_semantics=("arbitrary",)),
    )


def _pool_body(z_ref, bt3_ref, st_ref, g_ref, be_ref, w1_ref, b1_ref,
               w2r_ref, b2_ref, o_ref, accp, accc):
    i = pl.program_id(0)

    @pl.when(i == 0)
    def _():
        accp[...] = jnp.zeros_like(accp)
        accc[...] = jnp.zeros_like(accc)

    bt = bt3_ref[0]
    gi = lax.broadcasted_iota(jnp.int32, (_G, _RB), 0)
    oh = (gi == bt).astype(jnp.float32)
    accp[...] += lax.dot_general(oh, z_ref[...], (((1,), (0,)), ((), ())),
                                 preferred_element_type=jnp.float32)
    accc[...] += jnp.broadcast_to(jnp.sum(oh, axis=1, keepdims=True), (_G, _H))

    @pl.when(i == pl.num_programs(0) - 1)
    def _():
        a, b = _stats_to_affine(st_ref, g_ref[0], be_ref[0])
        pooled = accp[...] / jnp.maximum(accc[...], 1.0)
        pooled = pooled * a + b
        r1 = jnp.maximum(
            jnp.dot(pooled, w1_ref[...], preferred_element_type=jnp.float32) + b1_ref[...], 0.0)
        o_ref[...] = jnp.sum(r1 * w2r_ref[...], axis=1, keepdims=True) + b2_ref[...]


@functools.cache
def _pool_call(l):
    del l
    lrow = pl.BlockSpec((1, 1, _H), lambda i: (0, 0, 0))
    return pl.pallas_call(
        _pool_body,
        grid=(_NRB,),
        in_specs=[
            pl.BlockSpec((_RB, _H), lambda i: (i, 0)),
            pl.BlockSpec((1, 1, _RB), lambda i: (i, 0, 0)),
            pl.BlockSpec((8, _H), lambda i: (0, 0)),
            lrow,
            lrow,
            pl.BlockSpec((_H, _H), lambda i: (0, 0)),
            pl.BlockSpec((1, _H), lambda i: (0, 0)),
            pl.BlockSpec((1, _H), lambda i: (0, 0)),
            pl.BlockSpec((1, 1), lambda i: (0, 0)),
        ],
        out_specs=pl.BlockSpec((_G, 1), lambda i: (0, 0)),
        out_shape=jax.ShapeDtypeStruct((_G, 1), jnp.float32),
        scratch_shapes=[
            pltpu.VMEM((_G, _H), jnp.float32),
            pltpu.VMEM((_G, _H), jnp.float32),
        ],
        compiler_params=pltpu.CompilerParams(
            dimension_semantics=("arbitrary",)),
    )


def kernel(x, edge_index, batch, W1s, b1s, W2s, b2s, gammas, betas, eps_vec,
           lin1_W, lin1_b, lin2_W, lin2_b):
    src = edge_index[0]
    dst = edge_index[1]
    batch3 = batch.reshape(_NRB, 1, _RB)
    scal_rows = (1.0 + eps_vec)[:, None] * jnp.ones((1, _H), jnp.float32)
    num_layers = W1s.shape[0]
    b1s3 = b1s.reshape(num_layers, 1, _H)
    b2s3 = b2s.reshape(num_layers, 1, _H)
    gam3 = gammas.reshape(num_layers, 1, _H)
    bet3 = betas.reshape(num_layers, 1, _H)

    h = x
    out = None
    for l in range(num_layers):
        p0, p1 = _make_sc_segsum()(h, src, dst)
        z2, st = _mlp_call(0)(h, p0, p1, scal_rows[l:l + 1], W1s[l:l + 1],
                              b1s3[l:l + 1], W2s[l:l + 1], b2s3[l:l + 1])
        if l < num_layers - 1:
            h = _aff_call(0)(z2, st, gam3[l:l + 1], bet3[l:l + 1])
        else:
            out = _pool_call(0)(z2, batch3, st, gam3[l:l + 1], bet3[l:l + 1],
                                lin1_W, lin1_b.reshape(1, _H),
                                lin2_W.reshape(1, _H), lin2_b.reshape(1, 1))
    return out
